# baseline (speedup 1.0000x reference)
import jax
import jax.numpy as jnp
from jax import lax
from jax.experimental import pallas as pl
from jax.experimental.pallas import tpu as pltpu
from jax.experimental.pallas import tpu_sc as plsc

N = 10000
E = 320000
G = 64
NC, NS = 2, 16
KCH = 80
KE = 32
EPS = 20480
NIT = EPS // KE
NQ = NIT // 4
NROW = 10240
NH = NROW // 2
NPAD = NROW
RB = 640
NB = NROW // RB

_f32 = jnp.float32
_i32 = jnp.int32


def _sds(shape, dtype=_f32):
    return jax.ShapeDtypeStruct(shape, dtype)


def _deg_body(col_hbm, deg2_hbm, colv, onesv, fillv, deg_sp):
    c = lax.axis_index("c")
    s = lax.axis_index("s")
    w = c * NS + s
    for j in range(KCH // 16):
        onesv[pl.ds(j * 16, 16)] = jnp.ones((16,), _f32)
    ini = jnp.where(c == 0, 1.0, 0.0).astype(_f32)
    for j in range(640 // 16):
        fillv[pl.ds(j * 16, 16)] = jnp.zeros((16,), _f32) + ini
    pltpu.sync_copy(fillv, deg_sp.at[pl.ds(s * 640, 640)])
    pltpu.sync_copy(col_hbm.at[w], colv)
    plsc.subcore_barrier()

    def it(i, carry):
        pltpu.sync_copy(onesv, deg_sp.at[colv.at[i]], add=True)
        return carry

    lax.fori_loop(0, (E // 32) // KCH, it, 0)
    plsc.subcore_barrier()
    pltpu.sync_copy(deg_sp.at[pl.ds(s * 640, 640)],
                    deg2_hbm.at[pl.ds(c * NPAD + s * 640, 640)])


def _deg_call(col32):
    mesh = plsc.VectorSubcoreMesh(core_axis_name="c", subcore_axis_name="s")
    f = pl.kernel(
        _deg_body,
        out_type=_sds((2 * NPAD,)),
        mesh=mesh,
        scratch_types=[
            pltpu.VMEM((E // 32 // KCH, KCH), _i32),
            pltpu.VMEM((KCH,), _f32),
            pltpu.VMEM((640,), _f32),
            pltpu.VMEM_SHARED((NPAD,), _f32),
        ],
    )
    return f(col32)


def _mp_body(hs_hbm, row_hbm, col_hbm, z_hbm, rowv, colv, sb0, sb1, sb2, sb3,
             gbuf, ysp, semg0, semg1, semg2, semg3, semr, semc):
    c = lax.axis_index("c")
    s = lax.axis_index("s")
    npps = 2 * NH // NS
    base = c * 2 * NH
    pltpu.sync_copy(hs_hbm.at[pl.ds(base + s * npps, npps)],
                    ysp.at[pl.ds(s * npps, npps)])
    plsc.subcore_barrier()
    semg = (semg0, semg1, semg2, semg3)
    sb = (sb0, sb1, sb2, sb3)

    def idx_fire(q, p):
        pltpu.async_copy(row_hbm.at[s, pl.ds(4 * q, 4)], rowv.at[p], semr)
        pltpu.async_copy(col_hbm.at[s, pl.ds(4 * q, 4)], colv.at[p], semc)

    def idx_drain(q, p):
        pltpu.make_async_copy(row_hbm.at[s, pl.ds(4 * q, 4)], rowv.at[p],
                              semr).wait()
        pltpu.make_async_copy(col_hbm.at[s, pl.ds(4 * q, 4)], colv.at[p],
                              semc).wait()

    def build_scidx(p):
        for b in range(4):
            for jj in range(2 * KE // 16):
                sl = pl.ds(jj * 16, 16)
                lc = colv[p, b, sl] - base
                ok = (lc >= 0) & (lc < 2 * NH)
                sb[b][sl] = jnp.where(ok, lc, 2 * NH)

    def fire(p, b):
        pltpu.async_copy(hs_hbm.at[rowv.at[p, b]], gbuf.at[b], semg[b])

    def drain(p, b):
        pltpu.make_async_copy(hs_hbm.at[rowv.at[p, b]], gbuf.at[b],
                              semg[b]).wait()

    def scat(b):
        pltpu.sync_copy(gbuf.at[b], ysp.at[sb[b]], add=True)

    idx_fire(0, 0)
    idx_drain(0, 0)
    build_scidx(0)
    for b in range(4):
        fire(0, b)
    idx_fire(1, 1)

    def it(qq, carry):
        for p in range(2):
            q = 2 * qq + p
            pn = 1 - p
            qn = jnp.minimum(q + 1, NQ - 1)
            idx_drain(qn, pn)
            for b in range(4):
                drain(p, b)
                scat(b)
                fire(pn, b)
            build_scidx(pn)
            idx_fire(jnp.minimum(q + 2, NQ - 1), p)
        return carry

    lax.fori_loop(0, NQ // 2, it, 0)
    idx_drain(NQ - 1, 0)
    for b in range(4):
        drain(0, b)
    plsc.subcore_barrier()
    pltpu.sync_copy(ysp.at[pl.ds(s * npps, npps)],
                    z_hbm.at[pl.ds(base + s * npps, npps)])


def _mp_call(hs, row16, col16):
    mesh = plsc.VectorSubcoreMesh(core_axis_name="c", subcore_axis_name="s")
    f = pl.kernel(
        _mp_body,
        out_type=_sds((2 * NROW, 128)),
        mesh=mesh,
        scratch_types=[
            pltpu.VMEM((2, 4, 2 * KE), _i32),
            pltpu.VMEM((2, 4, 2 * KE), _i32),
            pltpu.VMEM((2 * KE,), _i32),
            pltpu.VMEM((2 * KE,), _i32),
            pltpu.VMEM((2 * KE,), _i32),
            pltpu.VMEM((2 * KE,), _i32),
            pltpu.VMEM((4, 2 * KE, 128), _f32),
            pltpu.VMEM_SHARED((2 * NH + 16, 128), _f32),
            pltpu.SemaphoreType.DMA,
            pltpu.SemaphoreType.DMA,
            pltpu.SemaphoreType.DMA,
            pltpu.SemaphoreType.DMA,
            pltpu.SemaphoreType.DMA,
            pltpu.SemaphoreType.DMA,
        ],
    )
    return f(hs, row16, col16)


def _mm1_body(x_ref, w_ref, deg_ref, o_ref):
    dinv = lax.rsqrt(deg_ref[...])
    h = jnp.dot(x_ref[...], w_ref[...], preferred_element_type=_f32,
                precision=lax.Precision.HIGHEST)
    o_ref[...] = h * dinv


def _mm1_call(x, W1, deg):
    return pl.pallas_call(
        _mm1_body,
        grid=(NB,),
        in_specs=[
            pl.BlockSpec((RB, 128), lambda i: (i, 0)),
            pl.BlockSpec((128, 256), lambda i: (0, 0)),
            pl.BlockSpec((RB, 1), lambda i: (i, 0)),
        ],
        out_specs=pl.BlockSpec((RB, 256), lambda i: (i, 0)),
        out_shape=_sds((NROW, 256)),
    )(x, W1, deg)


def _post_body(z_ref, deg_ref, b_ref, post_ref, cs_ref, cq_ref):
    i = pl.program_id(0)
    dinv = lax.rsqrt(deg_ref[...])
    y = jnp.maximum(z_ref[...] * dinv + b_ref[...], 0.0)
    post_ref[...] = y
    rowid = i * RB + lax.broadcasted_iota(_i32, (RB, 1), 0)
    ym = jnp.where(rowid < N, y, 0.0)
    cs = jnp.sum(ym, axis=0, keepdims=True)
    cq = jnp.sum(ym * ym, axis=0, keepdims=True)

    @pl.when(i == 0)
    def _():
        cs_ref[...] = cs
        cq_ref[...] = cq

    @pl.when(i > 0)
    def _():
        cs_ref[...] += cs
        cq_ref[...] += cq


def _post_call(z, deg, b1x256):
    return pl.pallas_call(
        _post_body,
        grid=(NB,),
        in_specs=[
            pl.BlockSpec((RB, 256), lambda i: (i, 0)),
            pl.BlockSpec((RB, 1), lambda i: (i, 0)),
            pl.BlockSpec((1, 256), lambda i: (0, 0)),
        ],
        out_specs=[
            pl.BlockSpec((RB, 256), lambda i: (i, 0)),
            pl.BlockSpec((1, 256), lambda i: (0, 0)),
            pl.BlockSpec((1, 256), lambda i: (0, 0)),
        ],
        out_shape=[_sds((NROW, 256)), _sds((1, 256)), _sds((1, 256))],
    )(z, deg, b1x256)


def _mm2_body(p_ref, w2_ref, a1_ref, c1_ref, deg_ref, o_ref):
    w = w2_ref[...] * a1_ref[...]
    acc = jnp.dot(p_ref[...], w, preferred_element_type=_f32,
                  precision=lax.Precision.HIGHEST)
    r = jnp.dot(c1_ref[...], w2_ref[...], preferred_element_type=_f32,
                precision=lax.Precision.HIGHEST)
    o_ref[...] = (acc + r) * lax.rsqrt(deg_ref[...])


def _mm2_call(post, W2, a1, c1, deg):
    return pl.pallas_call(
        _mm2_body,
        grid=(NB,),
        in_specs=[
            pl.BlockSpec((RB, 256), lambda i: (i, 0)),
            pl.BlockSpec((256, 256), lambda i: (0, 0)),
            pl.BlockSpec((256, 1), lambda i: (0, 0)),
            pl.BlockSpec((1, 256), lambda i: (0, 0)),
            pl.BlockSpec((RB, 1), lambda i: (i, 0)),
        ],
        out_specs=pl.BlockSpec((RB, 256), lambda i: (i, 0)),
        out_shape=_sds((NROW, 256)),
    )(post, W2, a1, c1, deg)


def _post2_body(z_ref, deg_ref, b_ref, bt_ref, y2_ref, cs_ref, cq_ref,
                s_ref, cnt_ref):
    i = pl.program_id(0)
    dinv = lax.rsqrt(deg_ref[...])
    y = jnp.maximum(z_ref[...] * dinv + b_ref[...], 0.0)
    y2_ref[...] = y
    rowid = i * RB + lax.broadcasted_iota(_i32, (RB, 1), 0)
    ym = jnp.where(rowid < N, y, 0.0)
    cs = jnp.sum(ym, axis=0, keepdims=True)
    cq = jnp.sum(ym * ym, axis=0, keepdims=True)
    gids = lax.broadcasted_iota(_i32, (1, G), 1)
    maskf = (bt_ref[...] == gids).astype(_f32)
    sblk = lax.dot_general(maskf, y, (((0,), (0,)), ((), ())),
                           preferred_element_type=_f32,
                           precision=lax.Precision.HIGHEST)
    cb = jnp.sum(maskf, axis=0, keepdims=True)

    @pl.when(i == 0)
    def _():
        cs_ref[...] = cs
        cq_ref[...] = cq
        s_ref[...] = sblk
        cnt_ref[...] = cb

    @pl.when(i > 0)
    def _():
        cs_ref[...] += cs
        cq_ref[...] += cq
        s_ref[...] += sblk
        cnt_ref[...] += cb


def _post2_call(z2, deg, b2x256, batch_col):
    return pl.pallas_call(
        _post2_body,
        grid=(NB,),
        in_specs=[
            pl.BlockSpec((RB, 256), lambda i: (i, 0)),
            pl.BlockSpec((RB, 1), lambda i: (i, 0)),
            pl.BlockSpec((1, 256), lambda i: (0, 0)),
            pl.BlockSpec((RB, 1), lambda i: (i, 0)),
        ],
        out_specs=[
            pl.BlockSpec((RB, 256), lambda i: (i, 0)),
            pl.BlockSpec((1, 256), lambda i: (0, 0)),
            pl.BlockSpec((1, 256), lambda i: (0, 0)),
            pl.BlockSpec((G, 256), lambda i: (0, 0)),
            pl.BlockSpec((1, G), lambda i: (0, 0)),
        ],
        out_shape=[_sds((NROW, 256)), _sds((1, 256)), _sds((1, 256)),
                   _sds((G, 256)), _sds((1, G))],
    )(z2, deg, b2x256, batch_col)


def _poolmm_body(y2_ref, st_ref, ct_ref, mx_ref, mn_ref):
    gb = pl.program_id(0)
    T = 32
    neg = jnp.full((T, 128), -jnp.inf, _f32)
    pos = jnp.full((T, 128), jnp.inf, _f32)

    for gg in range(8):
        g = gb * 8 + gg
        s = st_ref[0, g]
        n = ct_ref[0, g]
        s8 = pl.multiple_of((s // 8) * 8, 8)

        def tbody(i, carry):
            mx0, mx1, mn0, mn1 = carry
            b0 = pl.multiple_of(s8 + i * T, 8)
            r0 = y2_ref[pl.ds(b0, T), 0:128]
            r1 = y2_ref[pl.ds(b0, T), 128:256]
            ridx0 = b0 + lax.broadcasted_iota(_i32, (T, 1), 0)
            m0 = (ridx0 >= s) & (ridx0 < s + n)
            mx0 = jnp.maximum(mx0, jnp.where(m0, r0, -jnp.inf))
            mx1 = jnp.maximum(mx1, jnp.where(m0, r1, -jnp.inf))
            mn0 = jnp.minimum(mn0, jnp.where(m0, r0, jnp.inf))
            mn1 = jnp.minimum(mn1, jnp.where(m0, r1, jnp.inf))
            return mx0, mx1, mn0, mn1

        nt = (n + s - s8 + T - 1) // T
        mx0, mx1, mn0, mn1 = lax.fori_loop(0, nt, tbody,
                                           (neg, neg, pos, pos))
        mx_ref[gg:gg + 1, :] = jnp.concatenate(
            [jnp.max(mx0, axis=0, keepdims=True),
             jnp.max(mx1, axis=0, keepdims=True)], axis=1)
        mn_ref[gg:gg + 1, :] = jnp.concatenate(
            [jnp.min(mn0, axis=0, keepdims=True),
             jnp.min(mn1, axis=0, keepdims=True)], axis=1)


def _poolmm_call(y2, starts, cnts):
    return pl.pallas_call(
        _poolmm_body,
        grid=(G // 8,),
        in_specs=[
            pl.BlockSpec((NROW, 256), lambda g: (0, 0)),
            pl.BlockSpec(memory_space=pltpu.SMEM),
            pl.BlockSpec(memory_space=pltpu.SMEM),
        ],
        out_specs=[
            pl.BlockSpec((8, 256), lambda g: (g, 0)),
            pl.BlockSpec((8, 256), lambda g: (g, 0)),
        ],
        out_shape=[_sds((G, 256)), _sds((G, 256))],
    )(y2, starts, cnts)


def _head_body(s_ref, mx_ref, mn_ref, cnt_ref, a2_ref, c2_ref, wl_ref,
               bl_ref, o_ref):
    a2 = a2_ref[...]
    c2 = c2_ref[...]
    cnt = cnt_ref[...]
    x0 = s_ref[...] * a2 + cnt * c2
    x1 = x0 / jnp.maximum(cnt, 1.0)
    x2 = jnp.where(a2 > 0, mx_ref[...] * a2 + c2,
                   jnp.where(a2 < 0, mn_ref[...] * a2 + c2, c2))
    hp = lax.Precision.HIGHEST
    out = (jnp.dot(x0, wl_ref[0:256, :], preferred_element_type=_f32,
                   precision=hp)
           + jnp.dot(x1, wl_ref[256:512, :], preferred_element_type=_f32,
                     precision=hp)
           + jnp.dot(x2, wl_ref[512:768, :], preferred_element_type=_f32,
                     precision=hp))
    o_ref[...] = out + bl_ref[...]


def _head_call(S, mx, mn, cnt, a2, c2, Wl, bl):
    return pl.pallas_call(
        _head_body,
        out_shape=_sds((G, 128)),
    )(S, mx, mn, cnt, a2, c2, Wl, bl)


def kernel(x, edge_index, batch, W1, b1, W2, b2, g1, bt1, g2, bt2, Wl, bl):
    row = edge_index[0].astype(_i32)
    col = edge_index[1].astype(_i32)
    padi = jnp.full((NS, EPS - E // NS), N, _i32)
    rowp = jnp.concatenate([row.reshape(NS, E // NS), padi], axis=1)
    colp = jnp.concatenate([col.reshape(NS, E // NS), padi], axis=1)
    two = jnp.array([0, 1], _i32)
    row16 = (2 * rowp[..., None] + two).reshape(NS, NIT, 2 * KE)
    col16 = (2 * colp[..., None] + two).reshape(NS, NIT, 2 * KE)
    col32 = col.reshape(32, E // 32 // KCH, KCH)
    x_p = jnp.pad(x, ((0, NROW - N), (0, 0)))
    batch_col = jnp.pad(batch.astype(_i32), (0, NROW - N),
                        constant_values=G).reshape(NROW, 1)

    deg2 = _deg_call(col32)
    deg = (deg2[:NPAD] + deg2[NPAD:]).reshape(NROW, 1)

    hs1 = _mm1_call(x_p, W1, deg)
    z1 = _mp_call(hs1.reshape(2 * NROW, 128), row16,
                  col16).reshape(NROW, 256)
    post, cs1, cq1 = _post_call(z1, deg, b1.reshape(1, 256))

    mean1 = cs1.reshape(256) / N
    var1 = cq1.reshape(256) / N - mean1 * mean1
    a1 = g1 * lax.rsqrt(var1 + 1e-5)
    c1 = bt1 - mean1 * a1

    hs2 = _mm2_call(post, W2, a1.reshape(256, 1), c1.reshape(1, 256), deg)
    z2 = _mp_call(hs2.reshape(2 * NROW, 128), row16,
                  col16).reshape(NROW, 256)
    y2, cs2, cq2, S, cnt = _post2_call(z2, deg, b2.reshape(1, 256), batch_col)

    mean2 = cs2.reshape(256) / N
    var2 = cq2.reshape(256) / N - mean2 * mean2
    a2 = g2 * lax.rsqrt(var2 + 1e-5)
    c2 = bt2 - mean2 * a2

    cnt_i = cnt.reshape(G).astype(_i32)
    starts = jnp.concatenate(
        [jnp.zeros((1,), _i32), jnp.cumsum(cnt_i)[:-1]]).reshape(1, G)

    mx, mn = _poolmm_call(y2, starts, cnt_i.reshape(1, G))
    return _head_call(S, mx, mn, cnt.reshape(G, 1), a2.reshape(1, 256),
                      c2.reshape(1, 256), Wl, bl.reshape(1, 128))


# speedup vs baseline: 1.8783x; 1.8783x over previous
import functools

import jax
import jax.numpy as jnp
from jax import lax
from jax.experimental import pallas as pl
from jax.experimental.pallas import tpu as pltpu
from jax.experimental.pallas import tpu_sc as plsc

N = 10000
E = 320000
G = 64
NC, NS = 2, 16
KCH = 80
KE = 80
EPS = E // NS
NIT = EPS // KE
NROW = 10240
NPAD = NROW
RB = 640
NB = NROW // RB

_f32 = jnp.float32
_i32 = jnp.int32


def _sds(shape, dtype=_f32):
    return jax.ShapeDtypeStruct(shape, dtype)


def _deg_body(col_hbm, deg2_hbm, colv, onesv, fillv, deg_sp):
    c = lax.axis_index("c")
    s = lax.axis_index("s")
    w = c * NS + s
    for j in range(KCH // 16):
        onesv[pl.ds(j * 16, 16)] = jnp.ones((16,), _f32)
    ini = jnp.where(c == 0, 1.0, 0.0).astype(_f32)
    for j in range(640 // 16):
        fillv[pl.ds(j * 16, 16)] = jnp.zeros((16,), _f32) + ini
    pltpu.sync_copy(fillv, deg_sp.at[pl.ds(s * 640, 640)])
    pltpu.sync_copy(col_hbm.at[w], colv)
    plsc.subcore_barrier()

    def it(i, carry):
        pltpu.sync_copy(onesv, deg_sp.at[colv.at[i]], add=True)
        return carry

    lax.fori_loop(0, (E // 32) // KCH, it, 0)
    plsc.subcore_barrier()
    pltpu.sync_copy(deg_sp.at[pl.ds(s * 640, 640)],
                    deg2_hbm.at[pl.ds(c * NPAD + s * 640, 640)])


def _deg_call(col32):
    mesh = plsc.VectorSubcoreMesh(core_axis_name="c", subcore_axis_name="s")
    f = pl.kernel(
        _deg_body,
        out_type=_sds((2 * NPAD,)),
        mesh=mesh,
        scratch_types=[
            pltpu.VMEM((E // 32 // KCH, KCH), _i32),
            pltpu.VMEM((KCH,), _f32),
            pltpu.VMEM((640,), _f32),
            pltpu.VMEM_SHARED((NPAD,), _f32),
        ],
    )
    return f(col32)


def _mp_body(hs_hbm, row_hbm, col_hbm, z_hbm, rowv, colv, gidx, gbuf, ysp,
             sem):
    c = lax.axis_index("c")
    s = lax.axis_index("s")
    npps = NROW // NS
    pltpu.sync_copy(hs_hbm.at[pl.ds(c * NROW + s * npps, npps)],
                    ysp.at[pl.ds(s * npps, npps)])
    plsc.subcore_barrier()
    off = c * NROW

    def it(i, carry):
        pltpu.sync_copy(row_hbm.at[s, i], rowv)
        pltpu.sync_copy(col_hbm.at[s, i], colv)
        for j in range(KE // 16):
            sl = pl.ds(j * 16, 16)
            gidx[sl] = rowv[sl] + off
        pltpu.async_copy(hs_hbm.at[gidx], gbuf, sem).wait()
        pltpu.sync_copy(gbuf, ysp.at[colv], add=True)
        return carry

    lax.fori_loop(0, NIT, it, 0)
    plsc.subcore_barrier()
    pltpu.sync_copy(ysp.at[pl.ds(s * npps, npps)],
                    z_hbm.at[pl.ds(c * NROW + s * npps, npps)])


def _mp_call(hs, row16, col16):
    mesh = plsc.VectorSubcoreMesh(core_axis_name="c", subcore_axis_name="s")
    f = pl.kernel(
        _mp_body,
        out_type=_sds((2 * NROW, 128)),
        mesh=mesh,
        scratch_types=[
            pltpu.VMEM((KE,), _i32),
            pltpu.VMEM((KE,), _i32),
            pltpu.VMEM((KE,), _i32),
            pltpu.VMEM((KE, 128), _f32),
            pltpu.VMEM_SHARED((NROW, 128), _f32),
            pltpu.SemaphoreType.DMA,
        ],
    )
    return f(hs, row16, col16)


def _mm1_body(x_ref, w_ref, deg_ref, o_ref):
    dinv = lax.rsqrt(deg_ref[...])
    h = jnp.dot(x_ref[...], w_ref[...], preferred_element_type=_f32, precision=lax.Precision.HIGHEST)
    o_ref[...] = h * dinv


def _mm1_call(x, W1, deg):
    return pl.pallas_call(
        _mm1_body,
        grid=(2, NB),
        in_specs=[
            pl.BlockSpec((RB, 128), lambda h, i: (i, 0)),
            pl.BlockSpec((128, 128), lambda h, i: (0, h)),
            pl.BlockSpec((RB, 1), lambda h, i: (i, 0)),
        ],
        out_specs=pl.BlockSpec((RB, 128), lambda h, i: (h * NB + i, 0)),
        out_shape=_sds((2 * NROW, 128)),
    )(x, W1, deg)


def _post_body(z_ref, deg_ref, b_ref, post_ref, cs_ref, cq_ref):
    h = pl.program_id(0)
    i = pl.program_id(1)
    dinv = lax.rsqrt(deg_ref[...])
    y = jnp.maximum(z_ref[...] * dinv + b_ref[pl.ds(h, 1), :], 0.0)
    post_ref[...] = y
    rowid = i * RB + lax.broadcasted_iota(_i32, (RB, 1), 0)
    ym = jnp.where(rowid < N, y, 0.0)
    cs = jnp.sum(ym, axis=0, keepdims=True)
    cq = jnp.sum(ym * ym, axis=0, keepdims=True)

    @pl.when(i == 0)
    def _():
        cs_ref[pl.ds(h, 1), :] = cs
        cq_ref[pl.ds(h, 1), :] = cq

    @pl.when(i > 0)
    def _():
        cs_ref[pl.ds(h, 1), :] += cs
        cq_ref[pl.ds(h, 1), :] += cq


def _post_call(z, deg, b2x128):
    return pl.pallas_call(
        _post_body,
        grid=(2, NB),
        in_specs=[
            pl.BlockSpec((RB, 128), lambda h, i: (h * NB + i, 0)),
            pl.BlockSpec((RB, 1), lambda h, i: (i, 0)),
            pl.BlockSpec((2, 128), lambda h, i: (0, 0)),
        ],
        out_specs=[
            pl.BlockSpec((RB, 128), lambda h, i: (h * NB + i, 0)),
            pl.BlockSpec((2, 128), lambda h, i: (0, 0)),
            pl.BlockSpec((2, 128), lambda h, i: (0, 0)),
        ],
        out_shape=[_sds((2 * NROW, 128)), _sds((2, 128)), _sds((2, 128))],
    )(z, deg, b2x128)


def _mm2_body(pa_ref, pb_ref, w2a_ref, w2b_ref, a1a_ref, a1b_ref, c1_ref,
              deg_ref, o_ref):
    w0 = w2a_ref[...] * a1a_ref[...]
    w1 = w2b_ref[...] * a1b_ref[...]
    acc = (jnp.dot(pa_ref[...], w0, preferred_element_type=_f32, precision=lax.Precision.HIGHEST)
           + jnp.dot(pb_ref[...], w1, preferred_element_type=_f32, precision=lax.Precision.HIGHEST))
    r = (jnp.dot(c1_ref[:, 0:128], w2a_ref[...], preferred_element_type=_f32, precision=lax.Precision.HIGHEST)
         + jnp.dot(c1_ref[:, 128:256], w2b_ref[...],
                   preferred_element_type=_f32, precision=lax.Precision.HIGHEST))
    o_ref[...] = (acc + r) * lax.rsqrt(deg_ref[...])


def _mm2_call(post, W2, a1, c1, deg):
    return pl.pallas_call(
        _mm2_body,
        grid=(2, NB),
        in_specs=[
            pl.BlockSpec((RB, 128), lambda h, i: (i, 0)),
            pl.BlockSpec((RB, 128), lambda h, i: (NB + i, 0)),
            pl.BlockSpec((128, 128), lambda h, i: (0, h)),
            pl.BlockSpec((128, 128), lambda h, i: (1, h)),
            pl.BlockSpec((128, 1), lambda h, i: (0, 0)),
            pl.BlockSpec((128, 1), lambda h, i: (1, 0)),
            pl.BlockSpec((1, 256), lambda h, i: (0, 0)),
            pl.BlockSpec((RB, 1), lambda h, i: (i, 0)),
        ],
        out_specs=pl.BlockSpec((RB, 128), lambda h, i: (h * NB + i, 0)),
        out_shape=_sds((2 * NROW, 128)),
    )(post, post, W2, W2, a1, a1, c1, deg)


def _post2_body(z_ref, deg_ref, b_ref, bt_ref, y2_ref, cs_ref, cq_ref,
                s_ref, cnt_ref):
    h = pl.program_id(0)
    i = pl.program_id(1)
    dinv = lax.rsqrt(deg_ref[...])
    y = jnp.maximum(z_ref[...] * dinv + b_ref[pl.ds(h, 1), :], 0.0)
    y2_ref[...] = y
    rowid = i * RB + lax.broadcasted_iota(_i32, (RB, 1), 0)
    ym = jnp.where(rowid < N, y, 0.0)
    cs = jnp.sum(ym, axis=0, keepdims=True)
    cq = jnp.sum(ym * ym, axis=0, keepdims=True)
    gids = lax.broadcasted_iota(_i32, (1, G), 1)
    maskf = (bt_ref[...] == gids).astype(_f32)
    sblk = lax.dot_general(maskf, y, (((0,), (0,)), ((), ())),
                           preferred_element_type=_f32, precision=lax.Precision.HIGHEST)
    cb = jnp.sum(maskf, axis=0, keepdims=True)

    @pl.when(i == 0)
    def _():
        cs_ref[pl.ds(h, 1), :] = cs
        cq_ref[pl.ds(h, 1), :] = cq
        s_ref[...] = sblk

    @pl.when(i > 0)
    def _():
        cs_ref[pl.ds(h, 1), :] += cs
        cq_ref[pl.ds(h, 1), :] += cq
        s_ref[...] += sblk

    @pl.when((h == 0) & (i == 0))
    def _():
        cnt_ref[...] = cb

    @pl.when((h == 0) & (i > 0))
    def _():
        cnt_ref[...] += cb


def _post2_call(z2, deg, b2x128, batch_col):
    return pl.pallas_call(
        _post2_body,
        grid=(2, NB),
        in_specs=[
            pl.BlockSpec((RB, 128), lambda h, i: (h * NB + i, 0)),
            pl.BlockSpec((RB, 1), lambda h, i: (i, 0)),
            pl.BlockSpec((2, 128), lambda h, i: (0, 0)),
            pl.BlockSpec((RB, 1), lambda h, i: (i, 0)),
        ],
        out_specs=[
            pl.BlockSpec((RB, 128), lambda h, i: (h * NB + i, 0)),
            pl.BlockSpec((2, 128), lambda h, i: (0, 0)),
            pl.BlockSpec((2, 128), lambda h, i: (0, 0)),
            pl.BlockSpec((G, 128), lambda h, i: (0, h)),
            pl.BlockSpec((1, G), lambda h, i: (0, 0)),
        ],
        out_shape=[_sds((2 * NROW, 128)), _sds((2, 128)), _sds((2, 128)),
                   _sds((G, 256)), _sds((1, G))],
    )(z2, deg, b2x128, batch_col)


def _poolmm_body(y2_ref, st_ref, ct_ref, mx_ref, mn_ref):
    gb = pl.program_id(0)
    T = 32
    neg = jnp.full((T, 128), -jnp.inf, _f32)
    pos = jnp.full((T, 128), jnp.inf, _f32)

    for gg in range(8):
        g = gb * 8 + gg
        s = st_ref[0, g]
        n = ct_ref[0, g]

        def tbody(i, carry):
            mx0, mx1, mn0, mn1 = carry
            base = s + i * T
            b0 = jnp.minimum(base, 2 * NROW - T)
            b1 = jnp.minimum(base + NROW, 2 * NROW - T)
            r0 = y2_ref[pl.ds(b0, T), :]
            r1 = y2_ref[pl.ds(b1, T), :]
            ridx0 = b0 + lax.broadcasted_iota(_i32, (T, 1), 0)
            ridx1 = b1 + lax.broadcasted_iota(_i32, (T, 1), 0)
            m0 = (ridx0 >= s) & (ridx0 < s + n)
            m1 = (ridx1 >= NROW + s) & (ridx1 < NROW + s + n)
            mx0 = jnp.maximum(mx0, jnp.where(m0, r0, -jnp.inf))
            mx1 = jnp.maximum(mx1, jnp.where(m1, r1, -jnp.inf))
            mn0 = jnp.minimum(mn0, jnp.where(m0, r0, jnp.inf))
            mn1 = jnp.minimum(mn1, jnp.where(m1, r1, jnp.inf))
            return mx0, mx1, mn0, mn1

        nt = (n + T - 1) // T
        mx0, mx1, mn0, mn1 = lax.fori_loop(0, nt, tbody,
                                           (neg, neg, pos, pos))
        mx_ref[gg:gg + 1, :] = jnp.concatenate(
            [jnp.max(mx0, axis=0, keepdims=True),
             jnp.max(mx1, axis=0, keepdims=True)], axis=1)
        mn_ref[gg:gg + 1, :] = jnp.concatenate(
            [jnp.min(mn0, axis=0, keepdims=True),
             jnp.min(mn1, axis=0, keepdims=True)], axis=1)


def _poolmm_call(y2, starts, cnts):
    return pl.pallas_call(
        _poolmm_body,
        grid=(G // 8,),
        in_specs=[
            pl.BlockSpec((2 * NROW, 128), lambda g: (0, 0)),
            pl.BlockSpec(memory_space=pltpu.SMEM),
            pl.BlockSpec(memory_space=pltpu.SMEM),
        ],
        out_specs=[
            pl.BlockSpec((8, 256), lambda g: (g, 0)),
            pl.BlockSpec((8, 256), lambda g: (g, 0)),
        ],
        out_shape=[_sds((G, 256)), _sds((G, 256))],
    )(y2, starts, cnts)


def _head_body(s_ref, mx_ref, mn_ref, cnt_ref, a2_ref, c2_ref, wl_ref,
               bl_ref, o_ref):
    a2 = a2_ref[...]
    c2 = c2_ref[...]
    cnt = cnt_ref[...]
    x0 = s_ref[...] * a2 + cnt * c2
    x1 = x0 / jnp.maximum(cnt, 1.0)
    x2 = jnp.where(a2 > 0, mx_ref[...] * a2 + c2,
                   jnp.where(a2 < 0, mn_ref[...] * a2 + c2, c2))
    out = (jnp.dot(x0, wl_ref[0:256, :], preferred_element_type=_f32, precision=lax.Precision.HIGHEST)
           + jnp.dot(x1, wl_ref[256:512, :], preferred_element_type=_f32, precision=lax.Precision.HIGHEST)
           + jnp.dot(x2, wl_ref[512:768, :], preferred_element_type=_f32, precision=lax.Precision.HIGHEST))
    o_ref[...] = out + bl_ref[...]


def _head_call(S, mx, mn, cnt, a2, c2, Wl, bl):
    return pl.pallas_call(
        _head_body,
        out_shape=_sds((G, 128)),
    )(S, mx, mn, cnt, a2, c2, Wl, bl)


def kernel(x, edge_index, batch, W1, b1, W2, b2, g1, bt1, g2, bt2, Wl, bl):
    row = edge_index[0].astype(_i32)
    col = edge_index[1].astype(_i32)
    row16 = row.reshape(NS, NIT, KE)
    col16 = col.reshape(NS, NIT, KE)
    col32 = col.reshape(32, E // 32 // KCH, KCH)
    x_p = jnp.pad(x, ((0, NROW - N), (0, 0)))
    batch_col = jnp.pad(batch.astype(_i32), (0, NROW - N),
                        constant_values=G).reshape(NROW, 1)

    deg2 = _deg_call(col32)
    deg = (deg2[:NPAD] + deg2[NPAD:]).reshape(NROW, 1)

    hs1 = _mm1_call(x_p, W1, deg)
    z1 = _mp_call(hs1, row16, col16)
    post, cs1, cq1 = _post_call(z1, deg, b1.reshape(2, 128))

    mean1 = cs1.reshape(256) / N
    var1 = cq1.reshape(256) / N - mean1 * mean1
    a1 = g1 * lax.rsqrt(var1 + 1e-5)
    c1 = bt1 - mean1 * a1

    hs2 = _mm2_call(post, W2, a1.reshape(256, 1), c1.reshape(1, 256), deg)
    z2 = _mp_call(hs2, row16, col16)
    y2, cs2, cq2, S, cnt = _post2_call(z2, deg, b2.reshape(2, 128), batch_col)

    mean2 = cs2.reshape(256) / N
    var2 = cq2.reshape(256) / N - mean2 * mean2
    a2 = g2 * lax.rsqrt(var2 + 1e-5)
    c2 = bt2 - mean2 * a2

    cnt_i = cnt.reshape(G).astype(_i32)
    starts = jnp.concatenate(
        [jnp.zeros((1,), _i32), jnp.cumsum(cnt_i)[:-1]]).reshape(1, G)

    mx, mn = _poolmm_call(y2, starts, cnt_i.reshape(1, G))
    return _head_call(S, mx, mn, cnt.reshape(G, 1), a2.reshape(1, 256),
                      c2.reshape(1, 256), Wl, bl.reshape(1, 128))
